# baseline (speedup 1.0000x reference)
import functools

import jax
import jax.numpy as jnp
from jax import lax
from jax.experimental import pallas as pl
from jax.experimental.pallas import tpu as pltpu
from jax.experimental.pallas import tpu_sc as plsc

_B, _S, _D_MODEL, _D_SPACE = 4, 2048, 2048, 64
_N_GROUPS = 3
_N_PER_GROUP = 64
_N_TOTAL = _N_GROUPS * _N_PER_GROUP
_TOPK = (8, 4, 6)
_BS = 256
_NEG = -3.0e38
_GDN = lax.GatherDimensionNumbers(
    offset_dims=(), collapsed_slice_dims=(0,), start_index_map=(0,))


def _lane_perm(v, idx):
    return lax.gather(v, idx[:, None], _GDN, slice_sizes=(1,),
                      mode=lax.GatherScatterMode.PROMISE_IN_BOUNDS)


def _pool_body(x_ref, imp_ref, whi_ref, b_ref, emb_ref, out_ref, ehi_s):
    b_i = pl.program_id(0)
    s_i = pl.program_id(1)

    @pl.when((b_i == 0) & (s_i == 0))
    def _init_emb():
        emb = emb_ref[...]
        nrm = jnp.maximum(jnp.sqrt(jnp.sum(emb * emb, axis=1, keepdims=True)), 1e-12)
        ehi_s[...] = (emb / nrm).astype(jnp.bfloat16)

    @pl.when(s_i == 0)
    def _init_out():
        out_ref[...] = jnp.zeros_like(out_ref)

    xb = x_ref[0]
    imp = imp_ref[0, 0]
    xhi = xb.astype(jnp.bfloat16)
    h = jnp.dot(xhi, whi_ref[...], preferred_element_type=jnp.float32)
    h = h + b_ref[...]
    hhi = h.astype(jnp.bfloat16)
    cdims = (((1,), (1,)), ((), ()))
    logits = lax.dot_general(hhi, ehi_s[...], cdims,
                             preferred_element_type=jnp.float32)

    gid = lax.broadcasted_iota(jnp.int32, (_BS, _N_TOTAL), 1) // _N_PER_GROUP
    m0 = jnp.max(jnp.where(gid == 0, logits, _NEG), axis=-1, keepdims=True)
    m1 = jnp.max(jnp.where(gid == 1, logits, _NEG), axis=-1, keepdims=True)
    m2 = jnp.max(jnp.where(gid == 2, logits, _NEG), axis=-1, keepdims=True)
    mx = jnp.where(gid == 0, m0, jnp.where(gid == 1, m1, m2))
    e = jnp.exp(logits - mx)
    s0 = jnp.sum(jnp.where(gid == 0, e, 0.0), axis=-1, keepdims=True)
    s1 = jnp.sum(jnp.where(gid == 1, e, 0.0), axis=-1, keepdims=True)
    s2 = jnp.sum(jnp.where(gid == 2, e, 0.0), axis=-1, keepdims=True)
    ssum = jnp.where(gid == 0, s0, jnp.where(gid == 1, s1, s2))
    sm = e / ssum

    pooled = jnp.dot(imp.astype(jnp.bfloat16), sm.astype(jnp.bfloat16),
                     preferred_element_type=jnp.float32)
    out_ref[0] += pooled


def _topk_body(p_ref, cw_ref, qkw_ref, vw_ref):
    p = p_ref[:, 0, :]
    lane = lax.broadcasted_iota(jnp.int32, (_B, _N_TOTAL), 1)
    gidl = lane // _N_PER_GROUP
    kvec = jnp.where(gidl == 0, _TOPK[0],
                     jnp.where(gidl == 1, _TOPK[1], _TOPK[2]))
    rem = p
    keep = jnp.zeros_like(p)
    for i in range(max(_TOPK)):
        mg = [jnp.max(jnp.where(gidl == g, rem, _NEG), axis=-1, keepdims=True)
              for g in range(3)]
        mx = jnp.where(gidl == 0, mg[0],
                       jnp.where(gidl == 1, mg[1], mg[2]))
        cand = jnp.where(rem == mx, lane, _N_TOTAL)
        cg = [jnp.min(jnp.where(gidl == g, cand, _N_TOTAL), axis=-1,
                      keepdims=True) for g in range(3)]
        cidx = jnp.where(gidl == 0, cg[0],
                         jnp.where(gidl == 1, cg[1], cg[2]))
        sel = (lane == cidx) & (i < kvec)
        keep = jnp.where(sel, rem, keep)
        rem = jnp.where(sel, _NEG, rem)
    sg = [jnp.sum(jnp.where(gidl == g, keep, 0.0), axis=-1, keepdims=True)
          for g in range(3)]
    den = jnp.where(gidl == 0, sg[0],
                    jnp.where(gidl == 1, sg[1], sg[2])) + 1e-8
    sp = keep / den
    cw_ref[:, 0, :] = sp[:, :_N_PER_GROUP]
    qkw_ref[:, 0, :] = sp[:, _N_PER_GROUP:2 * _N_PER_GROUP]
    vw_ref[:, 0, :] = sp[:, 2 * _N_PER_GROUP:]


def _topk_tc_call(pooled):
    oshape = jax.ShapeDtypeStruct((_B, 1, _N_PER_GROUP), jnp.float32)
    return pl.pallas_call(
        _topk_body,
        out_shape=[oshape, oshape, oshape],
    )(pooled)


def _pool_call(x, importance, w_proj, b_proj, neuron_emb):
    grid = (_B, _S // _BS)
    return pl.pallas_call(
        _pool_body,
        grid=grid,
        in_specs=[
            pl.BlockSpec((1, _BS, _D_MODEL), lambda b, s: (b, s, 0)),
            pl.BlockSpec((1, 1, 1, _BS), lambda b, s: (b, s, 0, 0)),
            pl.BlockSpec((_D_MODEL, _D_SPACE), lambda b, s: (0, 0)),
            pl.BlockSpec((1, _D_SPACE), lambda b, s: (0, 0)),
            pl.BlockSpec((_N_TOTAL, _D_SPACE), lambda b, s: (0, 0)),
        ],
        out_specs=pl.BlockSpec((1, 1, _N_TOTAL), lambda b, s: (b, 0, 0)),
        out_shape=jax.ShapeDtypeStruct((_B, 1, _N_TOTAL), jnp.float32),
        scratch_shapes=[
            pltpu.VMEM((_N_TOTAL, _D_SPACE), jnp.bfloat16),
        ],
        compiler_params=pltpu.CompilerParams(
            dimension_semantics=("arbitrary", "arbitrary")),
    )(x, importance.reshape(_B, _S // _BS, 1, _BS),
      w_proj.astype(jnp.bfloat16),
      b_proj.reshape(1, -1), neuron_emb)


def _sc_topk_call(pooled_flat):
    mesh = plsc.VectorSubcoreMesh(core_axis_name="c", subcore_axis_name="s",
                                  num_cores=1)
    oshape = jax.ShapeDtypeStruct((_B * _N_PER_GROUP,), jnp.float32)

    @functools.partial(
        pl.kernel,
        mesh=mesh,
        out_type=(oshape, oshape, oshape, oshape),
        scratch_types=[pltpu.VMEM((_N_PER_GROUP,), jnp.float32)],
    )
    def sc_topk(pooled_hbm, cw_hbm, qw_hbm, kw_hbm, vw_hbm, row_v):
        wid = lax.axis_index("s")
        active = wid < _B * _N_GROUPS
        w = jnp.where(active, wid, 0)
        g = w % _N_GROUPS
        off = w * _N_PER_GROUP
        k = jnp.where(g == 0, _TOPK[0], jnp.where(g == 1, _TOPK[1], _TOPK[2]))
        pltpu.sync_copy(pooled_hbm.at[pl.ds(off, _N_PER_GROUP)], row_v)

        iota = lax.iota(jnp.int32, 16)
        chunks = [row_v[pl.ds(j * 16, 16)] for j in range(4)]
        outs = [jnp.zeros((16,), jnp.float32) for _ in range(4)]
        for i in range(max(_TOPK)):
            mv = chunks[0]
            mi = iota
            for j in range(1, 4):
                cv, cidx = chunks[j], iota + j * 16
                mi = jnp.where(cv > mv, cidx, mi)
                mv = jnp.where(cv > mv, cv, mv)
            for st in (1, 2, 4, 8):
                ov = _lane_perm(mv, iota ^ st)
                oi = _lane_perm(mi, iota ^ st)
                t = (jnp.where(ov > mv, 1, 0)
                     + jnp.where(ov == mv, 1, 0) * jnp.where(oi < mi, 1, 0))
                mi = jnp.where(t > 0, oi, mi)
                mv = jnp.where(t > 0, ov, mv)
            tgt = mi + jnp.where(i < k, 0, 1000)
            for j in range(4):
                sel = (iota + j * 16) == tgt
                outs[j] = jnp.where(sel, chunks[j], outs[j])
                chunks[j] = jnp.where(sel, _NEG, chunks[j])
        ssum = ((outs[0] + outs[1]) + (outs[2] + outs[3]))
        for st in (1, 2, 4, 8):
            ssum = ssum + _lane_perm(ssum, iota ^ st)
        scale = 1.0 / (ssum + 1e-8)
        for j in range(4):
            row_v[pl.ds(j * 16, 16)] = outs[j] * scale

        boff = (w // _N_GROUPS) * _N_PER_GROUP

        plsc.subcore_barrier()

        @pl.when(jnp.logical_and(active, g == 0))
        def _():
            pltpu.sync_copy(row_v, cw_hbm.at[pl.ds(boff, _N_PER_GROUP)])

        @pl.when(jnp.logical_and(active, g == 1))
        def _():
            pltpu.sync_copy(row_v, qw_hbm.at[pl.ds(boff, _N_PER_GROUP)])
            pltpu.sync_copy(row_v, kw_hbm.at[pl.ds(boff, _N_PER_GROUP)])

        @pl.when(jnp.logical_and(active, g == 2))
        def _():
            pltpu.sync_copy(row_v, vw_hbm.at[pl.ds(boff, _N_PER_GROUP)])

    return sc_topk(pooled_flat)


def _xla_topk(w, k):
    vals, idx = jax.lax.top_k(w, k)
    sparse = jnp.zeros_like(w).at[jnp.arange(w.shape[0])[:, None], idx].set(vals)
    return sparse / (sparse.sum(axis=-1, keepdims=True) + 1e-08)


def kernel(x, importance, W_proj, b_proj, neuron_emb):
    pooled = _pool_call(x, importance, W_proj, b_proj, neuron_emb)
    cw, qkw, vw = _topk_tc_call(pooled)
    shp = (_B, _N_PER_GROUP)
    return (cw.reshape(shp), qkw.reshape(shp), qkw.reshape(shp),
            vw.reshape(shp))


def _kernel_scvariant(x, importance, W_proj, b_proj, neuron_emb):
    pooled = _pool_call(x, importance, W_proj, b_proj, neuron_emb)
    cw, qw, kw, vw = _sc_topk_call(pooled.reshape(-1))
    shp = (_B, _N_PER_GROUP)
    return (cw.reshape(shp), qw.reshape(shp), kw.reshape(shp), vw.reshape(shp))


# speedup vs baseline: 1.4130x; 1.4130x over previous
import functools

import jax
import jax.numpy as jnp
from jax import lax
from jax.experimental import pallas as pl
from jax.experimental.pallas import tpu as pltpu
from jax.experimental.pallas import tpu_sc as plsc

_B, _S, _D_MODEL, _D_SPACE = 4, 2048, 2048, 64
_N_GROUPS = 3
_N_PER_GROUP = 64
_N_TOTAL = _N_GROUPS * _N_PER_GROUP
_TOPK = (8, 4, 6)
_BS = 1024
_NEG = -3.0e38
_GDN = lax.GatherDimensionNumbers(
    offset_dims=(), collapsed_slice_dims=(0,), start_index_map=(0,))


def _lane_perm(v, idx):
    return lax.gather(v, idx[:, None], _GDN, slice_sizes=(1,),
                      mode=lax.GatherScatterMode.PROMISE_IN_BOUNDS)


def _pool_body(x_ref, imp_ref, whi_ref, b_ref, emb_ref, out_ref, ehi_s):
    b_i = pl.program_id(0)
    s_i = pl.program_id(1)

    @pl.when((b_i == 0) & (s_i == 0))
    def _init_emb():
        emb = emb_ref[...]
        nrm = jnp.maximum(jnp.sqrt(jnp.sum(emb * emb, axis=1, keepdims=True)), 1e-12)
        ehi_s[...] = (emb / nrm).astype(jnp.bfloat16)

    @pl.when(s_i == 0)
    def _init_out():
        out_ref[...] = jnp.zeros_like(out_ref)

    xb = x_ref[0]
    imp = imp_ref[0, 0]
    xhi = xb.astype(jnp.bfloat16)
    h = jnp.dot(xhi, whi_ref[...], preferred_element_type=jnp.float32)
    h = h + b_ref[...]
    hhi = h.astype(jnp.bfloat16)
    cdims = (((1,), (1,)), ((), ()))
    logits = lax.dot_general(hhi, ehi_s[...], cdims,
                             preferred_element_type=jnp.float32)

    gid = lax.broadcasted_iota(jnp.int32, (_BS, _N_TOTAL), 1) // _N_PER_GROUP
    m0 = jnp.max(jnp.where(gid == 0, logits, _NEG), axis=-1, keepdims=True)
    m1 = jnp.max(jnp.where(gid == 1, logits, _NEG), axis=-1, keepdims=True)
    m2 = jnp.max(jnp.where(gid == 2, logits, _NEG), axis=-1, keepdims=True)
    mx = jnp.where(gid == 0, m0, jnp.where(gid == 1, m1, m2))
    e = jnp.exp(logits - mx)
    s0 = jnp.sum(jnp.where(gid == 0, e, 0.0), axis=-1, keepdims=True)
    s1 = jnp.sum(jnp.where(gid == 1, e, 0.0), axis=-1, keepdims=True)
    s2 = jnp.sum(jnp.where(gid == 2, e, 0.0), axis=-1, keepdims=True)
    ssum = jnp.where(gid == 0, s0, jnp.where(gid == 1, s1, s2))
    sm = e / ssum

    pooled = jnp.dot(imp.astype(jnp.bfloat16), sm.astype(jnp.bfloat16),
                     preferred_element_type=jnp.float32)
    out_ref[0] += pooled


def _topk_body(p_ref, cw_ref, qkw_ref, vw_ref):
    p = p_ref[:, 0, :]
    lane = lax.broadcasted_iota(jnp.int32, (_B, _N_TOTAL), 1)
    gidl = lane // _N_PER_GROUP
    kvec = jnp.where(gidl == 0, _TOPK[0],
                     jnp.where(gidl == 1, _TOPK[1], _TOPK[2]))
    rem = p
    keep = jnp.zeros_like(p)
    for i in range(max(_TOPK)):
        mg = [jnp.max(jnp.where(gidl == g, rem, _NEG), axis=-1, keepdims=True)
              for g in range(3)]
        mx = jnp.where(gidl == 0, mg[0],
                       jnp.where(gidl == 1, mg[1], mg[2]))
        cand = jnp.where(rem == mx, lane, _N_TOTAL)
        cg = [jnp.min(jnp.where(gidl == g, cand, _N_TOTAL), axis=-1,
                      keepdims=True) for g in range(3)]
        cidx = jnp.where(gidl == 0, cg[0],
                         jnp.where(gidl == 1, cg[1], cg[2]))
        sel = (lane == cidx) & (i < kvec)
        keep = jnp.where(sel, rem, keep)
        rem = jnp.where(sel, _NEG, rem)
    sg = [jnp.sum(jnp.where(gidl == g, keep, 0.0), axis=-1, keepdims=True)
          for g in range(3)]
    den = jnp.where(gidl == 0, sg[0],
                    jnp.where(gidl == 1, sg[1], sg[2])) + 1e-8
    sp = keep / den
    cw_ref[:, 0, :] = sp[:, :_N_PER_GROUP]
    qkw_ref[:, 0, :] = sp[:, _N_PER_GROUP:2 * _N_PER_GROUP]
    vw_ref[:, 0, :] = sp[:, 2 * _N_PER_GROUP:]


def _topk_tc_call(pooled):
    oshape = jax.ShapeDtypeStruct((_B, 1, _N_PER_GROUP), jnp.float32)
    return pl.pallas_call(
        _topk_body,
        out_shape=[oshape, oshape, oshape],
    )(pooled)


def _pool_call(x, importance, w_proj, b_proj, neuron_emb):
    grid = (_B, _S // _BS)
    return pl.pallas_call(
        _pool_body,
        grid=grid,
        in_specs=[
            pl.BlockSpec((1, _BS, _D_MODEL), lambda b, s: (b, s, 0)),
            pl.BlockSpec((1, 1, 1, _BS), lambda b, s: (b, s, 0, 0)),
            pl.BlockSpec((_D_MODEL, _D_SPACE), lambda b, s: (0, 0)),
            pl.BlockSpec((1, _D_SPACE), lambda b, s: (0, 0)),
            pl.BlockSpec((_N_TOTAL, _D_SPACE), lambda b, s: (0, 0)),
        ],
        out_specs=pl.BlockSpec((1, 1, _N_TOTAL), lambda b, s: (b, 0, 0)),
        out_shape=jax.ShapeDtypeStruct((_B, 1, _N_TOTAL), jnp.float32),
        scratch_shapes=[
            pltpu.VMEM((_N_TOTAL, _D_SPACE), jnp.bfloat16),
        ],
        compiler_params=pltpu.CompilerParams(
            dimension_semantics=("arbitrary", "arbitrary")),
    )(x, importance.reshape(_B, _S // _BS, 1, _BS),
      w_proj.astype(jnp.bfloat16),
      b_proj.reshape(1, -1), neuron_emb)


def _sc_topk_call(pooled_flat):
    mesh = plsc.VectorSubcoreMesh(core_axis_name="c", subcore_axis_name="s",
                                  num_cores=1)
    oshape = jax.ShapeDtypeStruct((_B * _N_PER_GROUP,), jnp.float32)

    @functools.partial(
        pl.kernel,
        mesh=mesh,
        out_type=(oshape, oshape, oshape, oshape),
        scratch_types=[pltpu.VMEM((_N_PER_GROUP,), jnp.float32)],
    )
    def sc_topk(pooled_hbm, cw_hbm, qw_hbm, kw_hbm, vw_hbm, row_v):
        wid = lax.axis_index("s")
        active = wid < _B * _N_GROUPS
        w = jnp.where(active, wid, 0)
        g = w % _N_GROUPS
        off = w * _N_PER_GROUP
        k = jnp.where(g == 0, _TOPK[0], jnp.where(g == 1, _TOPK[1], _TOPK[2]))
        pltpu.sync_copy(pooled_hbm.at[pl.ds(off, _N_PER_GROUP)], row_v)

        iota = lax.iota(jnp.int32, 16)
        chunks = [row_v[pl.ds(j * 16, 16)] for j in range(4)]
        outs = [jnp.zeros((16,), jnp.float32) for _ in range(4)]
        for i in range(max(_TOPK)):
            mv = chunks[0]
            mi = iota
            for j in range(1, 4):
                cv, cidx = chunks[j], iota + j * 16
                mi = jnp.where(cv > mv, cidx, mi)
                mv = jnp.where(cv > mv, cv, mv)
            for st in (1, 2, 4, 8):
                ov = _lane_perm(mv, iota ^ st)
                oi = _lane_perm(mi, iota ^ st)
                t = (jnp.where(ov > mv, 1, 0)
                     + jnp.where(ov == mv, 1, 0) * jnp.where(oi < mi, 1, 0))
                mi = jnp.where(t > 0, oi, mi)
                mv = jnp.where(t > 0, ov, mv)
            tgt = mi + jnp.where(i < k, 0, 1000)
            for j in range(4):
                sel = (iota + j * 16) == tgt
                outs[j] = jnp.where(sel, chunks[j], outs[j])
                chunks[j] = jnp.where(sel, _NEG, chunks[j])
        ssum = ((outs[0] + outs[1]) + (outs[2] + outs[3]))
        for st in (1, 2, 4, 8):
            ssum = ssum + _lane_perm(ssum, iota ^ st)
        scale = 1.0 / (ssum + 1e-8)
        for j in range(4):
            row_v[pl.ds(j * 16, 16)] = outs[j] * scale

        boff = (w // _N_GROUPS) * _N_PER_GROUP

        plsc.subcore_barrier()

        @pl.when(jnp.logical_and(active, g == 0))
        def _():
            pltpu.sync_copy(row_v, cw_hbm.at[pl.ds(boff, _N_PER_GROUP)])

        @pl.when(jnp.logical_and(active, g == 1))
        def _():
            pltpu.sync_copy(row_v, qw_hbm.at[pl.ds(boff, _N_PER_GROUP)])
            pltpu.sync_copy(row_v, kw_hbm.at[pl.ds(boff, _N_PER_GROUP)])

        @pl.when(jnp.logical_and(active, g == 2))
        def _():
            pltpu.sync_copy(row_v, vw_hbm.at[pl.ds(boff, _N_PER_GROUP)])

    return sc_topk(pooled_flat)


def _xla_topk(w, k):
    vals, idx = jax.lax.top_k(w, k)
    sparse = jnp.zeros_like(w).at[jnp.arange(w.shape[0])[:, None], idx].set(vals)
    return sparse / (sparse.sum(axis=-1, keepdims=True) + 1e-08)


def kernel(x, importance, W_proj, b_proj, neuron_emb):
    pooled = _pool_call(x, importance, W_proj, b_proj, neuron_emb)
    cw, qkw, vw = _topk_tc_call(pooled)
    shp = (_B, _N_PER_GROUP)
    return (cw.reshape(shp), qkw.reshape(shp), qkw.reshape(shp),
            vw.reshape(shp))


def _kernel_scvariant(x, importance, W_proj, b_proj, neuron_emb):
    pooled = _pool_call(x, importance, W_proj, b_proj, neuron_emb)
    cw, qw, kw, vw = _sc_topk_call(pooled.reshape(-1))
    shp = (_B, _N_PER_GROUP)
    return (cw.reshape(shp), qw.reshape(shp), kw.reshape(shp), vw.reshape(shp))
